# baseline (speedup 1.0000x reference)
import functools
import math

import jax
import jax.numpy as jnp
from jax import lax
from jax.experimental import pallas as pl
from jax.experimental.pallas import tpu as pltpu
from jax.experimental.pallas import tpu_sc as plsc

VOCAB = 1000000
D = 64
NSEQ = 16384
SL = 50
NC, NS = 2, 16
NW = NC * NS
IT_PW = NSEQ // 128 // NW
NUNIT = SL * IT_PW
SCALE = math.sqrt(D)


def _body(table_hbm, idx_hbm, out_hbm, idx_all, in_b, out_b,
          gsem0, gsem1, osem0, osem1):
    gsems = (gsem0, gsem1)
    osems = (osem0, osem1)
    w = lax.axis_index("s") * NC + lax.axis_index("c")

    pltpu.sync_copy(idx_hbm.at[:, pl.ds(w * 512, 512)], idx_all)

    def fire_gather(g, b):
        pltpu.make_async_copy(
            table_hbm.at[idx_all.at[g >> 2, pl.ds((g & 3) * 128, 128)]],
            in_b.at[b],
            gsems[b],
        ).start()

    def wait_gather(b):
        pltpu.make_async_copy(
            table_hbm.at[idx_all.at[0, pl.ds(0, 128)]],
            in_b.at[b],
            gsems[b],
        ).wait()

    lanes = lax.iota(jnp.int32, 16)
    drows = [lanes + c4 * 16 for c4 in range(D // 16)]
    zeros = jnp.zeros((16,), jnp.int32)

    fire_gather(0, 0)
    fire_gather(1, 1)

    def outer(i, _):
        for b in range(2):
            g = 2 * i + b
            j = g >> 2
            it = w * IT_PW + (g & 3)
            wait_gather(b)

            @pl.when(i >= 1)
            def _():
                pltpu.make_async_copy(
                    out_b.at[b],
                    out_hbm.at[j, :, pl.ds(it * 128, 128)],
                    osems[b],
                ).wait()

            @plsc.parallel_loop(0, 128, unroll=2)
            def _(r):
                col = zeros + r
                for c4 in range(D // 16):
                    plsc.store_scatter(
                        out_b.at[b], [drows[c4], col],
                        in_b[b, r, pl.ds(c4 * 16, 16)] * SCALE,
                    )

            @pl.when(i < NUNIT // 2 - 1)
            def _():
                fire_gather(g + 2, b)

            pltpu.make_async_copy(
                out_b.at[b],
                out_hbm.at[j, :, pl.ds(it * 128, 128)],
                osems[b],
            ).start()
        return 0

    lax.fori_loop(0, NUNIT // 2, outer, 0)

    for b in range(2):
        pltpu.make_async_copy(
            out_b.at[b], out_hbm.at[0, :, pl.ds(0, 128)], osems[b],
        ).wait()


@jax.jit
def kernel(x, table):
    table2 = jnp.pad(table, ((0, 0), (0, D)))
    idx_t = x.astype(jnp.int32).T
    mesh = plsc.VectorSubcoreMesh(
        core_axis_name="c", subcore_axis_name="s",
        num_cores=NC, num_subcores=NS,
    )
    fn = functools.partial(
        pl.kernel,
        out_type=jax.ShapeDtypeStruct((SL, D, NSEQ), jnp.float32),
        mesh=mesh,
        scratch_types=[
            pltpu.VMEM((SL, 512), jnp.int32),
            pltpu.VMEM((2, 128, 2 * D), jnp.float32),
            pltpu.VMEM((2, D, 128), jnp.float32),
            pltpu.SemaphoreType.DMA,
            pltpu.SemaphoreType.DMA,
            pltpu.SemaphoreType.DMA,
            pltpu.SemaphoreType.DMA,
        ],
        compiler_params=pltpu.CompilerParams(
            use_tc_tiling_on_sc=True, needs_layout_passes=False,
        ),
    )(_body)
    out = fn(table2, idx_t)
    return out.transpose(2, 0, 1)


# speedup vs baseline: 1.0829x; 1.0829x over previous
import functools
import math

import jax
import jax.numpy as jnp
from jax import lax
from jax.experimental import pallas as pl
from jax.experimental.pallas import tpu as pltpu
from jax.experimental.pallas import tpu_sc as plsc

VOCAB = 1000000
D = 64
NSEQ = 16384
SL = 50
NC, NS = 2, 16
NW = NC * NS
SEQ_PW = NSEQ // NW
CH_SEQ = 2
CH = CH_SEQ * SL
NCHUNK = SEQ_PW // CH_SEQ
SCALE = math.sqrt(D)


def _body(table_hbm, idx_hbm, out_hbm, idx_v, in_b, out_b,
          gsem0, gsem1, osem0, osem1):
    gsems = (gsem0, gsem1)
    osems = (osem0, osem1)
    w = lax.axis_index("s") * NC + lax.axis_index("c")

    pltpu.sync_copy(idx_hbm.at[w], idx_v)

    def prep_and_fire(g, b):
        pltpu.make_async_copy(
            table_hbm.at[idx_v.at[g]],
            in_b.at[b],
            gsems[b],
        ).start()

    def wait_gather(b):
        pltpu.make_async_copy(
            table_hbm.at[idx_v.at[0]],
            in_b.at[b],
            gsems[b],
        ).wait()

    prep_and_fire(0, 0)
    prep_and_fire(1, 1)

    def outer(i, _):
        for b in range(2):
            g = 2 * i + b
            seq0 = w * SEQ_PW + g * CH_SEQ
            wait_gather(b)

            @pl.when(i >= 1)
            def _():
                pltpu.make_async_copy(
                    out_b.at[b], out_hbm.at[pl.ds(seq0, CH_SEQ)], osems[b],
                ).wait()

            for si in range(CH_SEQ):
                @plsc.parallel_loop(0, SL, unroll=2)
                def _(r):
                    for c4 in range(D // 16):
                        out_b[b, si, r, pl.ds(c4 * 16, 16)] = (
                            in_b[b, si * SL + r, pl.ds(c4 * 16, 16)]
                            * SCALE
                        )

            @pl.when(i < NCHUNK // 2 - 1)
            def _():
                prep_and_fire(g + 2, b)

            pltpu.make_async_copy(
                out_b.at[b], out_hbm.at[pl.ds(seq0, CH_SEQ)], osems[b],
            ).start()
        return 0

    lax.fori_loop(0, NCHUNK // 2, outer, 0)

    for b in range(2):
        pltpu.make_async_copy(
            out_b.at[b], out_hbm.at[pl.ds(0, CH_SEQ)], osems[b],
        ).wait()


@jax.jit
def kernel(x, table):
    table2 = jnp.pad(table, ((0, 0), (0, D)))
    idx = x.reshape(-1).astype(jnp.int32).reshape(NW, NCHUNK, CH)
    mesh = plsc.VectorSubcoreMesh(
        core_axis_name="c", subcore_axis_name="s",
        num_cores=NC, num_subcores=NS,
    )
    fn = functools.partial(
        pl.kernel,
        out_type=jax.ShapeDtypeStruct((NSEQ, SL, D), jnp.float32),
        mesh=mesh,
        scratch_types=[
            pltpu.VMEM((NCHUNK, CH), jnp.int32),
            pltpu.VMEM((2, CH, 2 * D), jnp.float32),
            pltpu.VMEM((2, CH_SEQ, SL, D), jnp.float32),
            pltpu.SemaphoreType.DMA,
            pltpu.SemaphoreType.DMA,
            pltpu.SemaphoreType.DMA,
            pltpu.SemaphoreType.DMA,
        ],
        compiler_params=pltpu.CompilerParams(use_tc_tiling_on_sc=True),
    )(_body)
    return fn(table2, idx)
